# baseline (speedup 1.0000x reference)
import jax
import jax.numpy as jnp
from jax import lax
from jax.experimental import pallas as pl
from jax.experimental.pallas import tpu as pltpu
from jax.experimental.pallas import tpu_sc as plsc

N_NODES = 10000
N_EDGES = 320000
D = 128

NUM_CORES = 2
NUM_SUBCORES = 16
NW = NUM_CORES * NUM_SUBCORES
B_PER_W = N_EDGES // NW
CHUNK = 200
N_FULL = B_PER_W // CHUNK
NBUF = 2


def _gather_kernel(x_hbm, idx_hbm, out_hbm, idx_v, buf0, buf1, shm,
                   gsem0, gsem1, wsem0, wsem1, dsem0, dsem1):
    bufs = (buf0, buf1)
    gsems = (gsem0, gsem1)
    wsems = (wsem0, wsem1)
    dsems = (dsem0, dsem1)

    sid = lax.axis_index("s")
    wid = sid * NUM_CORES + lax.axis_index("c")
    base = wid * B_PER_W

    pltpu.sync_copy(idx_hbm.at[pl.ds(base, B_PER_W)], idx_v)

    def start_gather(g, b):
        pltpu.async_copy(
            x_hbm.at[idx_v.at[pl.ds(g * CHUNK, CHUNK)]], bufs[b], gsems[b])

    def wait_gather(b):
        pltpu.make_async_copy(
            x_hbm.at[idx_v.at[pl.ds(0, CHUNK)]], bufs[b], gsems[b]).wait()

    def start_write(g, b):
        pltpu.async_copy(
            bufs[b], out_hbm.at[pl.ds(base + g * CHUNK, CHUNK)], wsems[b])

    def start_relay_write(b, t):
        pltpu.async_copy(bufs[b], shm.at[sid, t], wsems[b])

    def wait_write(b):
        pltpu.make_async_copy(
            bufs[b], out_hbm.at[pl.ds(base, CHUNK)], wsems[b]).wait()

    def start_dma(g, t):
        pltpu.async_copy(
            shm.at[sid, t], out_hbm.at[pl.ds(base + g * CHUNK, CHUNK)],
            dsems[t])

    def wait_dma(t):
        pltpu.make_async_copy(
            shm.at[sid, t], out_hbm.at[pl.ds(base, CHUNK)], dsems[t]).wait()

    start_gather(0, 0)

    def body(j, carry):
        for b in range(NBUF):
            g = j * 2 + b
            bn = (b + 1) % NBUF

            @pl.when(g + 1 < N_FULL)
            def _():
                @pl.when(g >= 1)
                def _():
                    wait_write(bn)
                if b == 0:
                    @pl.when(j >= 1)
                    def _():
                        for t in range(2):
                            @pl.when((j - 1) % 2 == t)
                            def _():
                                start_dma(g - 1, t)
                start_gather(g + 1, bn)

            wait_gather(b)
            if b == 0:
                start_write(g, b)
            else:
                for t in range(2):
                    @pl.when(j % 2 == t)
                    def _():
                        @pl.when(j >= 2)
                        def _():
                            wait_dma(t)
                        start_relay_write(b, t)
        return carry

    lax.fori_loop(0, N_FULL // 2, body, 0)

    wait_write(0)
    wait_write(1)
    start_dma(N_FULL - 1, 0)
    wait_dma(1)
    wait_dma(0)


@jax.jit
def _gather(x, idx):
    mesh = plsc.VectorSubcoreMesh(core_axis_name="c", subcore_axis_name="s")
    return pl.kernel(
        _gather_kernel,
        out_type=jax.ShapeDtypeStruct((N_EDGES, D), jnp.float32),
        mesh=mesh,
        compiler_params=pltpu.CompilerParams(
            disable_bounds_checks=True, disable_semaphore_checks=True),
        scratch_types=[
            pltpu.VMEM((B_PER_W,), jnp.int32),
            pltpu.VMEM((CHUNK, D), jnp.float32),
            pltpu.VMEM((CHUNK, D), jnp.float32),
            pltpu.VMEM_SHARED((NUM_SUBCORES, 2, CHUNK, D), jnp.float32),
            pltpu.SemaphoreType.DMA,
            pltpu.SemaphoreType.DMA,
            pltpu.SemaphoreType.DMA,
            pltpu.SemaphoreType.DMA,
            pltpu.SemaphoreType.DMA,
            pltpu.SemaphoreType.DMA,
        ],
    )(x, idx)


def kernel(x, edge_index):
    return _gather(x, edge_index.astype(jnp.int32).reshape(-1))


# speedup vs baseline: 1.0031x; 1.0031x over previous
import jax
import jax.numpy as jnp
from jax import lax
from jax.experimental import pallas as pl
from jax.experimental.pallas import tpu as pltpu
from jax.experimental.pallas import tpu_sc as plsc

N_NODES = 10000
N_EDGES = 320000
D = 128

NUM_CORES = 2
NUM_SUBCORES = 16
NW = NUM_CORES * NUM_SUBCORES
B_PER_W = N_EDGES // NW
CHUNK = 200
N_FULL = B_PER_W // CHUNK
NBUF = 2


def _gather_kernel(x_hbm, idx_hbm, out_hbm, idx_v, buf0, buf1, shm,
                   gsem0, gsem1, wsem0, wsem1, dsem0, dsem1):
    bufs = (buf0, buf1)
    gsems = (gsem0, gsem1)
    wsems = (wsem0, wsem1)
    dsems = (dsem0, dsem1)

    sid = lax.axis_index("s")
    wid = sid * NUM_CORES + lax.axis_index("c")
    base = wid * B_PER_W

    pltpu.sync_copy(idx_hbm.at[pl.ds(base, B_PER_W)], idx_v)

    def start_gather(g, b):
        pltpu.async_copy(
            x_hbm.at[idx_v.at[pl.ds(g * CHUNK, CHUNK)]], bufs[b], gsems[b])

    def wait_gather(b):
        pltpu.make_async_copy(
            x_hbm.at[idx_v.at[pl.ds(0, CHUNK)]], bufs[b], gsems[b]).wait()

    def start_write(g, b):
        pltpu.async_copy(
            bufs[b], out_hbm.at[pl.ds(base + g * CHUNK, CHUNK)], wsems[b])

    def start_relay_write(b, t):
        pltpu.async_copy(bufs[b], shm.at[sid, t], wsems[b])

    def wait_write(b):
        pltpu.make_async_copy(
            bufs[b], out_hbm.at[pl.ds(base, CHUNK)], wsems[b]).wait()

    def start_dma(g, t):
        pltpu.async_copy(
            shm.at[sid, t], out_hbm.at[pl.ds(base + g * CHUNK, CHUNK)],
            dsems[t])

    def wait_dma(t):
        pltpu.make_async_copy(
            shm.at[sid, t], out_hbm.at[pl.ds(base, CHUNK)], dsems[t]).wait()

    start_gather(0, 0)

    def body(j, carry):
        for b in range(NBUF):
            g = j * 2 + b
            bn = (b + 1) % NBUF

            @pl.when(g + 1 < N_FULL)
            def _():
                @pl.when(g >= 1)
                def _():
                    wait_write(bn)
                if b == 0:
                    @pl.when(j >= 1)
                    def _():
                        for t in range(2):
                            @pl.when((j - 1) % 2 == t)
                            def _():
                                start_dma(g - 1, t)
                start_gather(g + 1, bn)

            wait_gather(b)
            if b == 0:
                start_write(g, b)
            else:
                for t in range(2):
                    @pl.when(j % 2 == t)
                    def _():
                        @pl.when(j >= 2)
                        def _():
                            wait_dma(t)
                        start_relay_write(b, t)
        return carry

    lax.fori_loop(0, N_FULL // 2, body, 0)

    wait_write(0)
    wait_write(1)
    start_dma(N_FULL - 1, 0)
    wait_dma(1)
    wait_dma(0)


@jax.jit
def _gather(x, idx):
    mesh = plsc.VectorSubcoreMesh(core_axis_name="c", subcore_axis_name="s")
    return pl.kernel(
        _gather_kernel,
        out_type=jax.ShapeDtypeStruct((N_EDGES, D), jnp.float32),
        mesh=mesh,
        scratch_types=[
            pltpu.VMEM((B_PER_W,), jnp.int32),
            pltpu.VMEM((CHUNK, D), jnp.float32),
            pltpu.VMEM((CHUNK, D), jnp.float32),
            pltpu.VMEM_SHARED((NUM_SUBCORES, 2, CHUNK, D), jnp.float32),
            pltpu.SemaphoreType.DMA,
            pltpu.SemaphoreType.DMA,
            pltpu.SemaphoreType.DMA,
            pltpu.SemaphoreType.DMA,
            pltpu.SemaphoreType.DMA,
            pltpu.SemaphoreType.DMA,
        ],
    )(x, idx)


def kernel(x, edge_index):
    return _gather(x, edge_index.astype(jnp.int32).reshape(-1))


# speedup vs baseline: 1.4818x; 1.4773x over previous
import jax
import jax.numpy as jnp
from jax import lax
from jax.experimental import pallas as pl
from jax.experimental.pallas import tpu as pltpu
from jax.experimental.pallas import tpu_sc as plsc

N_NODES = 10000
N_EDGES = 320000
D = 128

NUM_CORES = 2
NUM_SUBCORES = 16
NW = NUM_CORES * NUM_SUBCORES
B_PER_W = N_EDGES // NW
CHUNK = 80
N_FULL = B_PER_W // CHUNK
NBUF = 2


def _gather_kernel(x_hbm, idx_hbm, out_hbm, x_spm, idx_v, buf0, buf1,
                   gsem0, gsem1, wsem0, wsem1):
    bufs = (buf0, buf1)
    gsems = (gsem0, gsem1)
    wsems = (wsem0, wsem1)

    sid = lax.axis_index("s")
    wid = sid * NUM_CORES + lax.axis_index("c")
    base = wid * B_PER_W

    @pl.when(sid < 2)
    def _():
        pltpu.sync_copy(x_hbm.at[pl.ds(sid * 5000, 5000)],
                        x_spm.at[pl.ds(sid * 5000, 5000)])

    pltpu.sync_copy(idx_hbm.at[pl.ds(base, B_PER_W)], idx_v)
    plsc.subcore_barrier()

    def start_gather(g, b):
        pltpu.async_copy(
            x_spm.at[idx_v.at[pl.ds(g * CHUNK, CHUNK)]], bufs[b], gsems[b])

    def wait_gather(b):
        pltpu.make_async_copy(
            x_spm.at[idx_v.at[pl.ds(0, CHUNK)]], bufs[b], gsems[b]).wait()

    def start_write(g, b):
        pltpu.async_copy(
            bufs[b], out_hbm.at[pl.ds(base + g * CHUNK, CHUNK)], wsems[b])

    def wait_write(b):
        pltpu.make_async_copy(
            bufs[b], out_hbm.at[pl.ds(base, CHUNK)], wsems[b]).wait()

    start_gather(0, 0)

    def body(j, carry):
        for b in range(NBUF):
            g = j * 2 + b
            bn = (b + 1) % NBUF

            @pl.when(g + 1 < N_FULL)
            def _():
                @pl.when(g >= 1)
                def _():
                    wait_write(bn)
                start_gather(g + 1, bn)

            wait_gather(b)
            start_write(g, b)
        return carry

    lax.fori_loop(0, N_FULL // 2, body, 0)
    g_last = N_FULL - 1
    wait_gather(g_last % 2)
    start_write(g_last, g_last % 2)
    wait_write(1)
    wait_write(0)


@jax.jit
def _gather(x, idx):
    mesh = plsc.VectorSubcoreMesh(core_axis_name="c", subcore_axis_name="s")
    return pl.kernel(
        _gather_kernel,
        out_type=jax.ShapeDtypeStruct((N_EDGES, D), jnp.float32),
        mesh=mesh,
        scratch_types=[
            pltpu.VMEM_SHARED((N_NODES, D), jnp.float32),
            pltpu.VMEM((B_PER_W,), jnp.int32),
            pltpu.VMEM((CHUNK, D), jnp.float32),
            pltpu.VMEM((CHUNK, D), jnp.float32),
            pltpu.SemaphoreType.DMA,
            pltpu.SemaphoreType.DMA,
            pltpu.SemaphoreType.DMA,
            pltpu.SemaphoreType.DMA,
        ],
    )(x, idx)


def kernel(x, edge_index):
    return _gather(x, edge_index.astype(jnp.int32).reshape(-1))
